# baseline (speedup 1.0000x reference)
import functools

import jax
import jax.numpy as jnp
from jax import lax
from jax.experimental import pallas as pl
from jax.experimental.pallas import tpu as pltpu
from jax.experimental.pallas import tpu_sc as plsc

N = 10000
NPAD = 10240
E = 320000
EMB = 256
DH = EMB // 2
NTILES = 16
EPT = E // NTILES
K = 80
NPP = 5008
TRASH = NPP
ACC_ROWS = NPP + 8
WPT = 312
WLAST = NPP - 15 * WPT
ZR = 8
CAP = 20480


def _bin_body(src2, dst2, bsrc, bdst, ncnt, sflat, dflat, bs, bd, cnt_v):
    cid = lax.axis_index("c")
    sid = lax.axis_index("s")
    p = cid

    pltpu.sync_copy(src2.at[sid], sflat)
    pltpu.sync_copy(dst2.at[sid], dflat)

    zeros = jnp.zeros((16,), jnp.int32)
    trash = jnp.full((16,), TRASH, jnp.int32)

    def fbody(i, carry):
        bs[pl.ds(i * 16, 16)] = zeros
        bd[pl.ds(i * 16, 16)] = trash
        return carry
    lax.fori_loop(0, CAP // 16, fbody, 0)

    lo = p * NPP

    def sbody(i, off_v):
        d = dflat[pl.ds(i * 16, 16)] - lo
        s = sflat[pl.ds(i * 16, 16)]
        m = (d >= 0) & (d < NPP)
        cm = plsc.cumsum(m.astype(jnp.int32))
        pos = off_v + cm - 1
        plsc.store_scatter(bd, [pos], d, mask=m)
        plsc.store_scatter(bs, [pos], s, mask=m)
        return off_v + plsc.all_reduce_population_count(m)
    n_v = lax.fori_loop(0, EPT // 16, sbody, jnp.zeros((16,), jnp.int32))

    nc_v = (n_v + K - 1) // K
    nc_v = nc_v + (nc_v & 1)
    nc_v = jnp.maximum(nc_v, 2)
    nc0_v = (nc_v // 4) * 2
    lane = lax.iota(jnp.int32, 16)
    cnt_v[pl.ds(0, 16)] = jnp.where(lane == 1, nc0_v, nc_v)

    pltpu.sync_copy(cnt_v, ncnt.at[sid, p])
    pltpu.sync_copy(bs, bsrc.at[sid, p])
    pltpu.sync_copy(bd, bdst.at[sid, p])


_bin_call = functools.partial(
    pl.kernel,
    mesh=plsc.VectorSubcoreMesh(core_axis_name="c", subcore_axis_name="s"),
    compiler_params=pltpu.CompilerParams(needs_layout_passes=False),
    out_type=[
        jax.ShapeDtypeStruct((NTILES, 2, CAP), jnp.int32),
        jax.ShapeDtypeStruct((NTILES, 2, CAP), jnp.int32),
        jax.ShapeDtypeStruct((NTILES, 2, 16), jnp.int32),
    ],
    scratch_types=[
        pltpu.VMEM((EPT,), jnp.int32),
        pltpu.VMEM((EPT,), jnp.int32),
        pltpu.VMEM((CAP,), jnp.int32),
        pltpu.VMEM((CAP,), jnp.int32),
        pltpu.VMEM((16,), jnp.int32),
    ],
)(_bin_body)


BBLK = 2560


def _seg_sum_body(h0, h1, bsrc, bdst, ctl, out0, out1,
                  bs, bd, dst_adj, cnt_v, rows_a, rows_b, zero_v, acc,
                  sem_a, sem_b):
    cid = lax.axis_index("c")
    sid = lax.axis_index("s")

    pltpu.sync_copy(ctl.at[cid, sid], cnt_v)

    zeros16 = jnp.zeros((16,), jnp.float32)
    for r in range(ZR):
        for c in range(DH // 16):
            zero_v[r, pl.ds(c * 16, 16)] = zeros16

    def gather(j, rows_v, sem):
        @pl.when(cid == 0)
        def _g0():
            pltpu.async_copy(h0.at[bs.at[pl.ds(j * K, K)]], rows_v, sem)

        @pl.when(cid == 1)
        def _g1():
            pltpu.async_copy(h1.at[bs.at[pl.ds(j * K, K)]], rows_v, sem)

    def gwait(rows_v, sem):
        pltpu.make_async_copy(h0.at[pl.ds(0, K)], rows_v, sem).wait()

    def scatter_add(j, rows_v):
        for v in range(K // 16):
            dst_adj[pl.ds(v * 16, 16)] = bd[pl.ds(j * K + v * 16, 16)]
        pltpu.sync_copy(rows_v, acc.at[dst_adj], add=True)

    for p in range(2):
        pv = cnt_v[p, pl.ds(0, 16)]
        start = pv[0]
        cnt = pv[1]
        hi = start + cnt
        base_n = sid * WPT
        nz = jnp.where(sid == NTILES - 1, WLAST // ZR, WPT // ZR)

        def zbody(j, carry):
            pltpu.sync_copy(zero_v, acc.at[pl.ds(base_n + j * ZR, ZR)])
            return carry
        lax.fori_loop(0, nz, zbody, 0)

        b0 = (start * K) // BBLK
        bhi = (hi * K + BBLK - 1) // BBLK

        def lbody(b, carry):
            pltpu.sync_copy(bsrc.at[sid, p, pl.ds(b * BBLK, BBLK)],
                            bs.at[pl.ds(b * BBLK, BBLK)])
            pltpu.sync_copy(bdst.at[sid, p, pl.ds(b * BBLK, BBLK)],
                            bd.at[pl.ds(b * BBLK, BBLK)])
            return carry
        lax.fori_loop(b0, bhi, lbody, 0)

        plsc.subcore_barrier()

        @pl.when(cnt > 0)
        def _prologue():
            gather(start, rows_a, sem_a)
            gather(start + 1, rows_b, sem_b)

        def pair_body(jj, carry):
            j0 = start + 2 * jj
            gwait(rows_a, sem_a)
            scatter_add(j0, rows_a)

            @pl.when(j0 + 2 < hi)
            def _n0():
                gather(j0 + 2, rows_a, sem_a)

            gwait(rows_b, sem_b)
            scatter_add(j0 + 1, rows_b)

            @pl.when(j0 + 3 < hi)
            def _n1():
                gather(j0 + 3, rows_b, sem_b)
            return carry
        lax.fori_loop(0, cnt // 2, pair_body, 0)

        plsc.subcore_barrier()

        last = sid == NTILES - 1

        @pl.when((cid == 0) & jnp.logical_not(last))
        def _w0():
            pltpu.sync_copy(acc.at[pl.ds(base_n, WPT)],
                            out0.at[pl.ds(p * NPP + base_n, WPT)])

        @pl.when((cid == 0) & last)
        def _w0l():
            pltpu.sync_copy(acc.at[pl.ds(base_n, WLAST)],
                            out0.at[pl.ds(p * NPP + base_n, WLAST)])

        @pl.when((cid == 1) & jnp.logical_not(last))
        def _w1():
            pltpu.sync_copy(acc.at[pl.ds(base_n, WPT)],
                            out1.at[pl.ds(p * NPP + base_n, WPT)])

        @pl.when((cid == 1) & last)
        def _w1l():
            pltpu.sync_copy(acc.at[pl.ds(base_n, WLAST)],
                            out1.at[pl.ds(p * NPP + base_n, WLAST)])

        if p == 0:
            plsc.subcore_barrier()


_seg_sum_call = functools.partial(
    pl.kernel,
    mesh=plsc.VectorSubcoreMesh(core_axis_name="c", subcore_axis_name="s"),
    out_type=[
        jax.ShapeDtypeStruct((NPAD, DH), jnp.float32),
        jax.ShapeDtypeStruct((NPAD, DH), jnp.float32),
    ],
    scratch_types=[
        pltpu.VMEM((CAP,), jnp.int32),
        pltpu.VMEM((CAP,), jnp.int32),
        pltpu.VMEM((K,), jnp.int32),
        pltpu.VMEM((2, 16), jnp.int32),
        pltpu.VMEM((K, DH), jnp.float32),
        pltpu.VMEM((K, DH), jnp.float32),
        pltpu.VMEM((ZR, DH), jnp.float32),
        pltpu.VMEM_SHARED((ACC_ROWS, DH), jnp.float32),
        pltpu.SemaphoreType.DMA,
        pltpu.SemaphoreType.DMA,
    ],
)(_seg_sum_body)


def _segment_sum(h0, h1, bsrc, bdst, ctl):
    return _seg_sum_call(h0, h1, bsrc, bdst, ctl)


def _make_ctl(ncnt):
    nc = ncnt[:, :, 0]
    nc0 = ncnt[:, :, 1]
    z = jnp.zeros_like(nc)
    pad = jnp.zeros((NTILES, 2, 14), jnp.int32)
    full_c = jnp.concatenate([z[..., None], nc[..., None], pad], axis=2)
    full = jnp.stack([full_c, full_c], axis=0)
    c0 = jnp.concatenate([z[..., None], nc0[..., None], pad], axis=2)
    c1 = jnp.concatenate([nc0[..., None], (nc - nc0)[..., None], pad], axis=2)
    split = jnp.stack([c0, c1], axis=0)
    return full, split


RB = 10000


def _mlp_body(concat, *refs):
    if concat:
        (hl_ref, hh_ref, s0_ref, s1_ref, w1_ref, b1_ref, w2_ref, b2_ref,
         h2_ref, stats_ref) = refs
    else:
        (h_ref, s0_ref, s1_ref, w1_ref, b1_ref, w2_ref, b2_ref,
         h2_ref, stats_ref) = refs
    i = pl.program_id(0)
    if concat:
        agg = jnp.concatenate([hl_ref[...] + s0_ref[...],
                               hh_ref[...] + s1_ref[...]], axis=1)
    else:
        agg = h_ref[...] + s0_ref[...] + s1_ref[...]
    t = jnp.maximum(
        jax.lax.dot_general(agg, w1_ref[...], (((1,), (0,)), ((), ())),
                            preferred_element_type=jnp.float32)
        + b1_ref[...], 0.0)
    h2 = jax.lax.dot_general(t, w2_ref[...], (((1,), (0,)), ((), ())),
                             preferred_element_type=jnp.float32) + b2_ref[...]
    h2_ref[...] = h2
    s = jnp.sum(h2, axis=0, keepdims=True)
    sq = jnp.sum(h2 * h2, axis=0, keepdims=True)
    st = jnp.concatenate([s, sq], axis=0)

    @pl.when(i == 0)
    def _init():
        stats_ref[...] = st

    @pl.when(i != 0)
    def _acc():
        stats_ref[...] += st


def _mlp(h, s0, s1, w1, b1, w2, b2):
    concat = isinstance(h, tuple)
    d = 2 * DH if concat else h.shape[1]
    emb = w2.shape[1]
    if concat:
        h_specs = [pl.BlockSpec((RB, DH), lambda i: (i, 0)),
                   pl.BlockSpec((RB, DH), lambda i: (i, 0))]
        h_args = list(h)
    else:
        h_specs = [pl.BlockSpec((RB, d), lambda i: (i, 0))]
        h_args = [h]
    in_specs = h_specs + [
        pl.BlockSpec((RB, s0.shape[1]), lambda i: (i, 0)),
        pl.BlockSpec((RB, s1.shape[1]), lambda i: (i, 0)),
        pl.BlockSpec((d, emb), lambda i: (0, 0)),
        pl.BlockSpec((1, emb), lambda i: (0, 0)),
        pl.BlockSpec((emb, emb), lambda i: (0, 0)),
        pl.BlockSpec((1, emb), lambda i: (0, 0)),
    ]
    args = h_args + [s0, s1, w1, b1.reshape(1, emb), w2, b2.reshape(1, emb)]
    return pl.pallas_call(
        functools.partial(_mlp_body, concat),
        grid=(N // RB,),
        in_specs=in_specs,
        out_specs=[
            pl.BlockSpec((RB, emb), lambda i: (i, 0)),
            pl.BlockSpec((2, emb), lambda i: (0, 0)),
        ],
        out_shape=[
            jax.ShapeDtypeStruct((N, emb), jnp.float32),
            jax.ShapeDtypeStruct((2, emb), jnp.float32),
        ],
    )(*args)


def _bn_body(relu, split, *refs):
    if split:
        h2_ref, stats_ref, g_ref, be_ref, lo_ref, hi_ref = refs
    else:
        h2_ref, stats_ref, g_ref, be_ref, out_ref = refs
    inv_n = 1.0 / N
    mean = stats_ref[0:1, :] * inv_n
    var = stats_ref[1:2, :] * inv_n - mean * mean
    scale = jax.lax.rsqrt(var + 1e-5) * g_ref[...]
    y = (h2_ref[...] - mean) * scale + be_ref[...]
    if relu:
        y = jnp.maximum(y, 0.0)
    if split:
        lo_ref[...] = y[:, :DH]
        hi_ref[...] = y[:, DH:]
    else:
        out_ref[...] = y


def _bn(h2, stats, g, be, relu, split):
    emb = h2.shape[1]
    if split:
        out_specs = [pl.BlockSpec((RB, DH), lambda i: (i, 0)),
                     pl.BlockSpec((RB, DH), lambda i: (i, 0))]
        out_shape = [jax.ShapeDtypeStruct((N, DH), jnp.float32),
                     jax.ShapeDtypeStruct((N, DH), jnp.float32)]
    else:
        out_specs = pl.BlockSpec((RB, emb), lambda i: (i, 0))
        out_shape = jax.ShapeDtypeStruct((N, emb), jnp.float32)
    return pl.pallas_call(
        functools.partial(_bn_body, relu, split),
        grid=(N // RB,),
        in_specs=[
            pl.BlockSpec((RB, emb), lambda i: (i, 0)),
            pl.BlockSpec((2, emb), lambda i: (0, 0)),
            pl.BlockSpec((1, emb), lambda i: (0, 0)),
            pl.BlockSpec((1, emb), lambda i: (0, 0)),
        ],
        out_specs=out_specs,
        out_shape=out_shape,
    )(h2, stats, g.reshape(1, emb), be.reshape(1, emb))


def kernel(x, edge_index,
           W1_0, b1_0, W2_0, b2_0, gamma_0, beta_0,
           W1_1, b1_1, W2_1, b2_1, gamma_1, beta_1,
           W1_2, b1_2, W2_2, b2_2, gamma_2, beta_2):
    src2 = edge_index[0].reshape(NTILES, EPT)
    dst2 = edge_index[1].reshape(NTILES, EPT)
    bsrc, bdst, ncnt = _bin_call(src2, dst2)
    ctl_full, ctl_split = _make_ctl(ncnt)

    s0, s1 = _segment_sum(x, x, bsrc, bdst, ctl_split)
    h2, stats = _mlp(x, s0, s1, W1_0, b1_0, W2_0, b2_0)
    hlo, hhi = _bn(h2, stats, gamma_0, beta_0, relu=True, split=True)

    for (w1, b1, w2, b2, g, be, relu, split) in (
            (W1_1, b1_1, W2_1, b2_1, gamma_1, beta_1, True, True),
            (W1_2, b1_2, W2_2, b2_2, gamma_2, beta_2, False, False)):
        s0, s1 = _segment_sum(hlo, hhi, bsrc, bdst, ctl_full)
        h2, stats = _mlp((hlo, hhi), s0, s1, w1, b1, w2, b2)
        out = _bn(h2, stats, g, be, relu=relu, split=split)
        if split:
            hlo, hhi = out
    return out


# speedup vs baseline: 1.0134x; 1.0134x over previous
import functools

import jax
import jax.numpy as jnp
from jax import lax
from jax.experimental import pallas as pl
from jax.experimental.pallas import tpu as pltpu
from jax.experimental.pallas import tpu_sc as plsc

N = 10000
NPAD = 10240
E = 320000
EMB = 256
DH = EMB // 2
NTILES = 16
EPT = E // NTILES
K = 80
NPP = 5008
TRASH = NPP
ACC_ROWS = NPP + 8
WPT = 312
WLAST = NPP - 15 * WPT
ZR = 8
CAP = 20480


def _bin_body(src2, dst2, bsrc, bdst, ncnt, sflat, dflat, bs, bd, cnt_v):
    cid = lax.axis_index("c")
    sid = lax.axis_index("s")
    p = cid

    pltpu.sync_copy(src2.at[sid], sflat)
    pltpu.sync_copy(dst2.at[sid], dflat)

    zeros = jnp.zeros((16,), jnp.int32)
    trash = jnp.full((16,), TRASH, jnp.int32)

    def fbody(i, carry):
        bs[pl.ds(i * 16, 16)] = zeros
        bd[pl.ds(i * 16, 16)] = trash
        return carry
    lax.fori_loop(0, CAP // 16, fbody, 0)

    lo = p * NPP

    def sbody(i, off_v):
        d = dflat[pl.ds(i * 16, 16)] - lo
        s = sflat[pl.ds(i * 16, 16)]
        m = (d >= 0) & (d < NPP)
        cm = plsc.cumsum(m.astype(jnp.int32))
        pos = off_v + cm - 1
        plsc.store_scatter(bd, [pos], d, mask=m)
        plsc.store_scatter(bs, [pos], s, mask=m)
        return off_v + plsc.all_reduce_population_count(m)
    n_v = lax.fori_loop(0, EPT // 16, sbody, jnp.zeros((16,), jnp.int32))

    nc_v = (n_v + K - 1) // K
    nc_v = nc_v + (nc_v & 1)
    nc_v = jnp.maximum(nc_v, 2)
    nc0_v = (nc_v // 4) * 2
    lane = lax.iota(jnp.int32, 16)
    cnt_v[pl.ds(0, 16)] = jnp.where(lane == 1, nc0_v, nc_v)

    pltpu.sync_copy(cnt_v, ncnt.at[sid, p])
    pltpu.sync_copy(bs, bsrc.at[sid, p])
    pltpu.sync_copy(bd, bdst.at[sid, p])


_bin_call = functools.partial(
    pl.kernel,
    mesh=plsc.VectorSubcoreMesh(core_axis_name="c", subcore_axis_name="s"),
    compiler_params=pltpu.CompilerParams(needs_layout_passes=False),
    out_type=[
        jax.ShapeDtypeStruct((NTILES, 2, CAP), jnp.int32),
        jax.ShapeDtypeStruct((NTILES, 2, CAP), jnp.int32),
        jax.ShapeDtypeStruct((NTILES, 2, 16), jnp.int32),
    ],
    scratch_types=[
        pltpu.VMEM((EPT,), jnp.int32),
        pltpu.VMEM((EPT,), jnp.int32),
        pltpu.VMEM((CAP,), jnp.int32),
        pltpu.VMEM((CAP,), jnp.int32),
        pltpu.VMEM((16,), jnp.int32),
    ],
)(_bin_body)


BBLK = 2560


def _seg_sum_body(h0, h1, bsrc, bdst, ctl, out0, out1,
                  bs, bd, dst_adj, cnt_v, rows_a, rows_b, zero_v, acc,
                  sem_a, sem_b):
    cid = lax.axis_index("c")
    sid = lax.axis_index("s")

    pltpu.sync_copy(ctl.at[cid, sid], cnt_v)

    zeros16 = jnp.zeros((16,), jnp.float32)
    for r in range(ZR):
        for c in range(DH // 16):
            zero_v[r, pl.ds(c * 16, 16)] = zeros16

    def gather(j, rows_v, sem):
        @pl.when(cid == 0)
        def _g0():
            pltpu.async_copy(h0.at[bs.at[pl.ds(j * K, K)]], rows_v, sem)

        @pl.when(cid == 1)
        def _g1():
            pltpu.async_copy(h1.at[bs.at[pl.ds(j * K, K)]], rows_v, sem)

    def gwait(rows_v, sem):
        pltpu.make_async_copy(h0.at[pl.ds(0, K)], rows_v, sem).wait()

    def scatter_add(j, rows_v):
        for v in range(K // 16):
            dst_adj[pl.ds(v * 16, 16)] = bd[pl.ds(j * K + v * 16, 16)]
        pltpu.sync_copy(rows_v, acc.at[dst_adj], add=True)

    for p in range(2):
        pv = cnt_v[p, pl.ds(0, 16)]
        start = pv[0]
        cnt = pv[1]
        hi = start + cnt
        base_n = sid * WPT
        nz = jnp.where(sid == NTILES - 1, WLAST // ZR, WPT // ZR)

        def zbody(j, carry):
            pltpu.sync_copy(zero_v, acc.at[pl.ds(base_n + j * ZR, ZR)])
            return carry
        lax.fori_loop(0, nz, zbody, 0)

        b0 = (start * K) // BBLK
        bhi = (hi * K + BBLK - 1) // BBLK

        def lbody(b, carry):
            pltpu.sync_copy(bsrc.at[sid, p, pl.ds(b * BBLK, BBLK)],
                            bs.at[pl.ds(b * BBLK, BBLK)])
            pltpu.sync_copy(bdst.at[sid, p, pl.ds(b * BBLK, BBLK)],
                            bd.at[pl.ds(b * BBLK, BBLK)])
            return carry
        lax.fori_loop(b0, bhi, lbody, 0)

        plsc.subcore_barrier()

        @pl.when(cnt > 0)
        def _prologue():
            gather(start, rows_a, sem_a)
            gather(start + 1, rows_b, sem_b)

        def pair_body(jj, carry):
            j0 = start + 2 * jj
            gwait(rows_a, sem_a)
            scatter_add(j0, rows_a)

            @pl.when(j0 + 2 < hi)
            def _n0():
                gather(j0 + 2, rows_a, sem_a)

            gwait(rows_b, sem_b)
            scatter_add(j0 + 1, rows_b)

            @pl.when(j0 + 3 < hi)
            def _n1():
                gather(j0 + 3, rows_b, sem_b)
            return carry
        lax.fori_loop(0, cnt // 2, pair_body, 0)

        plsc.subcore_barrier()

        last = sid == NTILES - 1

        @pl.when((cid == 0) & jnp.logical_not(last))
        def _w0():
            pltpu.sync_copy(acc.at[pl.ds(base_n, WPT)],
                            out0.at[pl.ds(p * NPP + base_n, WPT)])

        @pl.when((cid == 0) & last)
        def _w0l():
            pltpu.sync_copy(acc.at[pl.ds(base_n, WLAST)],
                            out0.at[pl.ds(p * NPP + base_n, WLAST)])

        @pl.when((cid == 1) & jnp.logical_not(last))
        def _w1():
            pltpu.sync_copy(acc.at[pl.ds(base_n, WPT)],
                            out1.at[pl.ds(p * NPP + base_n, WPT)])

        @pl.when((cid == 1) & last)
        def _w1l():
            pltpu.sync_copy(acc.at[pl.ds(base_n, WLAST)],
                            out1.at[pl.ds(p * NPP + base_n, WLAST)])

        if p == 0:
            plsc.subcore_barrier()


_seg_sum_call = functools.partial(
    pl.kernel,
    mesh=plsc.VectorSubcoreMesh(core_axis_name="c", subcore_axis_name="s"),
    out_type=[
        jax.ShapeDtypeStruct((NPAD, DH), jnp.float32),
        jax.ShapeDtypeStruct((NPAD, DH), jnp.float32),
    ],
    scratch_types=[
        pltpu.VMEM((CAP,), jnp.int32),
        pltpu.VMEM((CAP,), jnp.int32),
        pltpu.VMEM((K,), jnp.int32),
        pltpu.VMEM((2, 16), jnp.int32),
        pltpu.VMEM((K, DH), jnp.float32),
        pltpu.VMEM((K, DH), jnp.float32),
        pltpu.VMEM((ZR, DH), jnp.float32),
        pltpu.VMEM_SHARED((ACC_ROWS, DH), jnp.float32),
        pltpu.SemaphoreType.DMA,
        pltpu.SemaphoreType.DMA,
    ],
)(_seg_sum_body)


def _segment_sum(h0, h1, bsrc, bdst, ctl):
    return _seg_sum_call(h0, h1, bsrc, bdst, ctl)


def _make_ctl(ncnt):
    nc = ncnt[:, :, 0]
    nc0 = ncnt[:, :, 1]
    z = jnp.zeros_like(nc)
    pad = jnp.zeros((NTILES, 2, 14), jnp.int32)
    full_c = jnp.concatenate([z[..., None], nc[..., None], pad], axis=2)
    full = jnp.stack([full_c, full_c], axis=0)
    c0 = jnp.concatenate([z[..., None], nc0[..., None], pad], axis=2)
    c1 = jnp.concatenate([nc0[..., None], (nc - nc0)[..., None], pad], axis=2)
    split = jnp.stack([c0, c1], axis=0)
    return full, split


RB = 5000


def _mlp_body(concat, *refs):
    if concat:
        (hl_ref, hh_ref, s0_ref, s1_ref, w1_ref, b1_ref, w2_ref, b2_ref,
         h2_ref, stats_ref) = refs
    else:
        (h_ref, s0_ref, s1_ref, w1_ref, b1_ref, w2_ref, b2_ref,
         h2_ref, stats_ref) = refs
    i = pl.program_id(0)
    if concat:
        agg = jnp.concatenate([hl_ref[...] + s0_ref[...],
                               hh_ref[...] + s1_ref[...]], axis=1)
    else:
        agg = h_ref[...] + s0_ref[...] + s1_ref[...]
    t = jnp.maximum(
        jax.lax.dot_general(agg, w1_ref[...], (((1,), (0,)), ((), ())),
                            preferred_element_type=jnp.float32)
        + b1_ref[...], 0.0)
    h2 = jax.lax.dot_general(t, w2_ref[...], (((1,), (0,)), ((), ())),
                             preferred_element_type=jnp.float32) + b2_ref[...]
    h2_ref[...] = h2
    s = jnp.sum(h2, axis=0, keepdims=True)
    sq = jnp.sum(h2 * h2, axis=0, keepdims=True)
    st = jnp.concatenate([s, sq], axis=0)

    @pl.when(i == 0)
    def _init():
        stats_ref[...] = st

    @pl.when(i != 0)
    def _acc():
        stats_ref[...] += st


def _mlp(h, s0, s1, w1, b1, w2, b2):
    concat = isinstance(h, tuple)
    d = 2 * DH if concat else h.shape[1]
    emb = w2.shape[1]
    if concat:
        h_specs = [pl.BlockSpec((RB, DH), lambda i: (i, 0)),
                   pl.BlockSpec((RB, DH), lambda i: (i, 0))]
        h_args = list(h)
    else:
        h_specs = [pl.BlockSpec((RB, d), lambda i: (i, 0))]
        h_args = [h]
    in_specs = h_specs + [
        pl.BlockSpec((RB, s0.shape[1]), lambda i: (i, 0)),
        pl.BlockSpec((RB, s1.shape[1]), lambda i: (i, 0)),
        pl.BlockSpec((d, emb), lambda i: (0, 0)),
        pl.BlockSpec((1, emb), lambda i: (0, 0)),
        pl.BlockSpec((emb, emb), lambda i: (0, 0)),
        pl.BlockSpec((1, emb), lambda i: (0, 0)),
    ]
    args = h_args + [s0, s1, w1, b1.reshape(1, emb), w2, b2.reshape(1, emb)]
    return pl.pallas_call(
        functools.partial(_mlp_body, concat),
        grid=(N // RB,),
        in_specs=in_specs,
        out_specs=[
            pl.BlockSpec((RB, emb), lambda i: (i, 0)),
            pl.BlockSpec((2, emb), lambda i: (0, 0)),
        ],
        out_shape=[
            jax.ShapeDtypeStruct((N, emb), jnp.float32),
            jax.ShapeDtypeStruct((2, emb), jnp.float32),
        ],
    )(*args)


def _bn_body(relu, split, *refs):
    if split:
        h2_ref, stats_ref, g_ref, be_ref, lo_ref, hi_ref = refs
    else:
        h2_ref, stats_ref, g_ref, be_ref, out_ref = refs
    inv_n = 1.0 / N
    mean = stats_ref[0:1, :] * inv_n
    var = stats_ref[1:2, :] * inv_n - mean * mean
    scale = jax.lax.rsqrt(var + 1e-5) * g_ref[...]
    y = (h2_ref[...] - mean) * scale + be_ref[...]
    if relu:
        y = jnp.maximum(y, 0.0)
    if split:
        lo_ref[...] = y[:, :DH]
        hi_ref[...] = y[:, DH:]
    else:
        out_ref[...] = y


def _bn(h2, stats, g, be, relu, split):
    emb = h2.shape[1]
    if split:
        out_specs = [pl.BlockSpec((RB, DH), lambda i: (i, 0)),
                     pl.BlockSpec((RB, DH), lambda i: (i, 0))]
        out_shape = [jax.ShapeDtypeStruct((N, DH), jnp.float32),
                     jax.ShapeDtypeStruct((N, DH), jnp.float32)]
    else:
        out_specs = pl.BlockSpec((RB, emb), lambda i: (i, 0))
        out_shape = jax.ShapeDtypeStruct((N, emb), jnp.float32)
    return pl.pallas_call(
        functools.partial(_bn_body, relu, split),
        grid=(N // RB,),
        in_specs=[
            pl.BlockSpec((RB, emb), lambda i: (i, 0)),
            pl.BlockSpec((2, emb), lambda i: (0, 0)),
            pl.BlockSpec((1, emb), lambda i: (0, 0)),
            pl.BlockSpec((1, emb), lambda i: (0, 0)),
        ],
        out_specs=out_specs,
        out_shape=out_shape,
    )(h2, stats, g.reshape(1, emb), be.reshape(1, emb))


def kernel(x, edge_index,
           W1_0, b1_0, W2_0, b2_0, gamma_0, beta_0,
           W1_1, b1_1, W2_1, b2_1, gamma_1, beta_1,
           W1_2, b1_2, W2_2, b2_2, gamma_2, beta_2):
    src2 = edge_index[0].reshape(NTILES, EPT)
    dst2 = edge_index[1].reshape(NTILES, EPT)
    bsrc, bdst, ncnt = _bin_call(src2, dst2)
    ctl_full, ctl_split = _make_ctl(ncnt)

    s0, s1 = _segment_sum(x, x, bsrc, bdst, ctl_split)
    h2, stats = _mlp(x, s0, s1, W1_0, b1_0, W2_0, b2_0)
    hlo, hhi = _bn(h2, stats, gamma_0, beta_0, relu=True, split=True)

    for (w1, b1, w2, b2, g, be, relu, split) in (
            (W1_1, b1_1, W2_1, b2_1, gamma_1, beta_1, True, True),
            (W1_2, b1_2, W2_2, b2_2, gamma_2, beta_2, False, False)):
        s0, s1 = _segment_sum(hlo, hhi, bsrc, bdst, ctl_full)
        h2, stats = _mlp((hlo, hhi), s0, s1, w1, b1, w2, b2)
        out = _bn(h2, stats, g, be, relu=relu, split=split)
        if split:
            hlo, hhi = out
    return out
